# Initial kernel scaffold; baseline (speedup 1.0000x reference)
import jax
import jax.numpy as jnp
from jax.experimental import pallas as pl


def kernel(x, w1, b1, w2, b2):
    raise NotImplementedError("write your pallas kernel here")


# speedup vs baseline: 3.1092x; 3.1092x over previous
import functools

import jax
import jax.numpy as jnp
from jax.experimental import pallas as pl
from jax.experimental.pallas import tpu as pltpu


def _gate_kernel(x_ref, w1_ref, b1_ref, w2_ref, b2_ref, o_ref, *,
                 s_true, needs_mask):
    x = x_ref[...].astype(jnp.float32)

    if needs_mask:
        lane = jax.lax.broadcasted_iota(jnp.int32, x.shape, 2)
        x_for_max = jnp.where(lane < s_true, x, -jnp.inf)
    else:
        x_for_max = x

    avg = jnp.sum(x, axis=-1) * (1.0 / s_true)
    mx = jnp.max(x_for_max, axis=-1)

    w1 = w1_ref[...]
    b1 = b1_ref[...]
    w2 = w2_ref[...]
    b2 = b2_ref[...]

    def mlp(p):
        h = jnp.maximum(
            jnp.dot(p, w1, preferred_element_type=jnp.float32) + b1, 0.0)
        return jnp.dot(h, w2, preferred_element_type=jnp.float32) + b2

    scale = jax.nn.sigmoid(mlp(avg) + mlp(mx))
    o_ref[...] = (x * scale[:, :, None]).astype(o_ref.dtype)


def kernel(x, w1, b1, w2, b2):
    B, C, D, H, W = x.shape
    S = D * H * W
    x3 = x.reshape(B, C, S)

    LANE = 128
    s_pad = -(-S // LANE) * LANE
    if s_pad != S:
        x3 = jnp.pad(x3, ((0, 0), (0, 0), (0, s_pad - S)))

    BT = 4
    while B % BT != 0:
        BT //= 2
    grid = (B // BT,)

    out3 = pl.pallas_call(
        functools.partial(_gate_kernel, s_true=S, needs_mask=(s_pad != S)),
        out_shape=jax.ShapeDtypeStruct((B, C, s_pad), x.dtype),
        grid=grid,
        in_specs=[
            pl.BlockSpec((BT, C, s_pad), lambda i: (i, 0, 0)),
            pl.BlockSpec(w1.shape, lambda i: (0, 0)),
            pl.BlockSpec(b1.shape, lambda i: (0, 0)),
            pl.BlockSpec(w2.shape, lambda i: (0, 0)),
            pl.BlockSpec(b2.shape, lambda i: (0, 0)),
        ],
        out_specs=pl.BlockSpec((BT, C, s_pad), lambda i: (i, 0, 0)),
        compiler_params=pltpu.CompilerParams(
            dimension_semantics=("parallel",)
        ),
    )(x3, w1, b1, w2, b2)

    if s_pad != S:
        out3 = out3[:, :, :S]
    return out3.reshape(B, C, D, H, W)


# speedup vs baseline: 3.2373x; 1.0412x over previous
import functools

import jax
import jax.numpy as jnp
from jax.experimental import pallas as pl
from jax.experimental.pallas import tpu as pltpu


def _gate_kernel(x_ref, w1_ref, b1_ref, w2_ref, b2_ref, o_ref, *,
                 s_true, needs_mask):
    x = x_ref[...].astype(jnp.float32)

    if needs_mask:
        lane = jax.lax.broadcasted_iota(jnp.int32, x.shape, 2)
        x_for_max = jnp.where(lane < s_true, x, -jnp.inf)
    else:
        x_for_max = x

    avg = jnp.sum(x, axis=-1) * (1.0 / s_true)
    mx = jnp.max(x_for_max, axis=-1)

    w1 = w1_ref[...]
    b1 = b1_ref[...]
    w2 = w2_ref[...]
    b2 = b2_ref[...]

    def mlp(p):
        h = jnp.maximum(
            jnp.dot(p, w1, preferred_element_type=jnp.float32) + b1, 0.0)
        return jnp.dot(h, w2, preferred_element_type=jnp.float32) + b2

    scale = jax.nn.sigmoid(mlp(avg) + mlp(mx))
    o_ref[...] = (x * scale[:, :, None]).astype(o_ref.dtype)


def kernel(x, w1, b1, w2, b2):
    B, C, D, H, W = x.shape
    S = D * H * W
    x3 = x.reshape(B, C, S)

    LANE = 128
    s_pad = -(-S // LANE) * LANE
    if s_pad != S:
        x3 = jnp.pad(x3, ((0, 0), (0, 0), (0, s_pad - S)))

    BT = 8
    while B % BT != 0:
        BT //= 2
    grid = (B // BT,)

    out3 = pl.pallas_call(
        functools.partial(_gate_kernel, s_true=S, needs_mask=(s_pad != S)),
        out_shape=jax.ShapeDtypeStruct((B, C, s_pad), x.dtype),
        grid=grid,
        in_specs=[
            pl.BlockSpec((BT, C, s_pad), lambda i: (i, 0, 0)),
            pl.BlockSpec(w1.shape, lambda i: (0, 0)),
            pl.BlockSpec(b1.shape, lambda i: (0, 0)),
            pl.BlockSpec(w2.shape, lambda i: (0, 0)),
            pl.BlockSpec(b2.shape, lambda i: (0, 0)),
        ],
        out_specs=pl.BlockSpec((BT, C, s_pad), lambda i: (i, 0, 0)),
        compiler_params=pltpu.CompilerParams(
            dimension_semantics=("parallel",)
        ),
    )(x3, w1, b1, w2, b2)

    if s_pad != S:
        out3 = out3[:, :, :S]
    return out3.reshape(B, C, D, H, W)


# speedup vs baseline: 3.2522x; 1.0046x over previous
import functools

import jax
import jax.numpy as jnp
from jax.experimental import pallas as pl
from jax.experimental.pallas import tpu as pltpu


def _gate_kernel(x_ref, w1_ref, b1_ref, w2_ref, b2_ref, o_ref, *,
                 s_true, needs_mask):
    x = x_ref[...].astype(jnp.float32)

    if needs_mask:
        lane = jax.lax.broadcasted_iota(jnp.int32, x.shape, 2)
        x_for_max = jnp.where(lane < s_true, x, -jnp.inf)
    else:
        x_for_max = x

    avg = jnp.sum(x, axis=-1) * (1.0 / s_true)
    mx = jnp.max(x_for_max, axis=-1)

    w1 = w1_ref[...]
    b1 = b1_ref[...]
    w2 = w2_ref[...]
    b2 = b2_ref[...]

    def mlp(p):
        h = jnp.maximum(
            jnp.dot(p, w1, preferred_element_type=jnp.float32) + b1, 0.0)
        return jnp.dot(h, w2, preferred_element_type=jnp.float32) + b2

    scale = jax.nn.sigmoid(mlp(avg) + mlp(mx))
    o_ref[...] = (x * scale[:, :, None]).astype(o_ref.dtype)


def kernel(x, w1, b1, w2, b2):
    B, C, D, H, W = x.shape
    S = D * H * W
    x3 = x.reshape(B, C, S)

    LANE = 128
    s_pad = -(-S // LANE) * LANE
    if s_pad != S:
        x3 = jnp.pad(x3, ((0, 0), (0, 0), (0, s_pad - S)))

    BT = 16
    while B % BT != 0:
        BT //= 2
    grid = (B // BT,)

    out3 = pl.pallas_call(
        functools.partial(_gate_kernel, s_true=S, needs_mask=(s_pad != S)),
        out_shape=jax.ShapeDtypeStruct((B, C, s_pad), x.dtype),
        grid=grid,
        in_specs=[
            pl.BlockSpec((BT, C, s_pad), lambda i: (i, 0, 0)),
            pl.BlockSpec(w1.shape, lambda i: (0, 0)),
            pl.BlockSpec(b1.shape, lambda i: (0, 0)),
            pl.BlockSpec(w2.shape, lambda i: (0, 0)),
            pl.BlockSpec(b2.shape, lambda i: (0, 0)),
        ],
        out_specs=pl.BlockSpec((BT, C, s_pad), lambda i: (i, 0, 0)),
        compiler_params=pltpu.CompilerParams(
            dimension_semantics=("parallel",)
        ),
    )(x3, w1, b1, w2, b2)

    if s_pad != S:
        out3 = out3[:, :, :S]
    return out3.reshape(B, C, D, H, W)


# speedup vs baseline: 5.8672x; 1.8041x over previous
import functools

import jax
import jax.numpy as jnp
from jax.experimental import pallas as pl
from jax.experimental.pallas import tpu as pltpu


def _pool_kernel_t(x_ref, sum_ref, max_ref):
    k = pl.program_id(1)
    x = x_ref[...].astype(jnp.float32)
    ps = jnp.sum(x, axis=1)
    pm = jnp.max(x, axis=1)

    @pl.when(k == 0)
    def _():
        sum_ref[0] = ps
        max_ref[0] = pm

    @pl.when(k != 0)
    def _():
        sum_ref[0] = sum_ref[0] + ps
        max_ref[0] = jnp.maximum(max_ref[0], pm)


def _apply_kernel_t(x_ref, psum_ref, pmax_ref, w1_ref, b1_ref, w2_ref,
                    b2_ref, o_ref, *, inv_s):
    s = jnp.sum(psum_ref[...], axis=0)
    m = jnp.max(pmax_ref[...], axis=0)
    avg = s * inv_s

    w1 = w1_ref[...]
    w2 = w2_ref[...]
    b1 = b1_ref[...].reshape(-1, 1)
    b2 = b2_ref[...].reshape(-1, 1)

    def mlp(p):
        h = jax.lax.dot_general(
            w1, p, (((0,), (0,)), ((), ())),
            preferred_element_type=jnp.float32)
        h = jnp.maximum(h + b1, 0.0)
        o = jax.lax.dot_general(
            w2, h, (((0,), (0,)), ((), ())),
            preferred_element_type=jnp.float32)
        return o + b2

    scale = jax.nn.sigmoid(mlp(avg) + mlp(m))
    o_ref[...] = (x_ref[...] * scale[:, None, :].astype(o_ref.dtype))


def _channel_gate_native(x3, w1, b1, w2, b2, S):
    B, C, _ = x3.shape
    xT = jnp.transpose(x3, (1, 2, 0))

    ST = next(t for t in (512, 256, 128, 64, 32, 16, 8) if S % t == 0)
    N = S // ST
    P = 2 if N % 2 == 0 else 1
    K = N // P

    psum, pmax = pl.pallas_call(
        _pool_kernel_t,
        out_shape=(
            jax.ShapeDtypeStruct((P, C, B), jnp.float32),
            jax.ShapeDtypeStruct((P, C, B), jnp.float32),
        ),
        grid=(P, K),
        in_specs=[pl.BlockSpec((C, ST, B), lambda p, k: (0, p * K + k, 0))],
        out_specs=(
            pl.BlockSpec((1, C, B), lambda p, k: (p, 0, 0)),
            pl.BlockSpec((1, C, B), lambda p, k: (p, 0, 0)),
        ),
        compiler_params=pltpu.CompilerParams(
            dimension_semantics=("parallel", "arbitrary")
        ),
    )(xT)

    outT = pl.pallas_call(
        functools.partial(_apply_kernel_t, inv_s=1.0 / S),
        out_shape=jax.ShapeDtypeStruct((C, S, B), x3.dtype),
        grid=(P, K),
        in_specs=[
            pl.BlockSpec((C, ST, B), lambda p, k: (0, p * K + k, 0)),
            pl.BlockSpec((P, C, B), lambda p, k: (0, 0, 0)),
            pl.BlockSpec((P, C, B), lambda p, k: (0, 0, 0)),
            pl.BlockSpec(w1.shape, lambda p, k: (0, 0)),
            pl.BlockSpec(b1.shape, lambda p, k: (0, 0)),
            pl.BlockSpec(w2.shape, lambda p, k: (0, 0)),
            pl.BlockSpec(b2.shape, lambda p, k: (0, 0)),
        ],
        out_specs=pl.BlockSpec((C, ST, B), lambda p, k: (0, p * K + k, 0)),
        compiler_params=pltpu.CompilerParams(
            dimension_semantics=("parallel", "parallel")
        ),
    )(xT, psum, pmax, w1, b1, w2, b2)

    return jnp.transpose(outT, (2, 0, 1))


def _gate_kernel(x_ref, w1_ref, b1_ref, w2_ref, b2_ref, o_ref, *,
                 s_true, needs_mask):
    x = x_ref[...].astype(jnp.float32)

    if needs_mask:
        lane = jax.lax.broadcasted_iota(jnp.int32, x.shape, 2)
        x_for_max = jnp.where(lane < s_true, x, -jnp.inf)
    else:
        x_for_max = x

    avg = jnp.sum(x, axis=-1) * (1.0 / s_true)
    mx = jnp.max(x_for_max, axis=-1)

    def mlp(p):
        h = jnp.maximum(
            jnp.dot(p, w1_ref[...], preferred_element_type=jnp.float32)
            + b1_ref[...], 0.0)
        return jnp.dot(h, w2_ref[...],
                       preferred_element_type=jnp.float32) + b2_ref[...]

    scale = jax.nn.sigmoid(mlp(avg) + mlp(mx))
    o_ref[...] = (x * scale[:, :, None]).astype(o_ref.dtype)


def _channel_gate_padded(x3, w1, b1, w2, b2, S):
    B, C, _ = x3.shape
    LANE = 128
    s_pad = -(-S // LANE) * LANE
    if s_pad != S:
        x3 = jnp.pad(x3, ((0, 0), (0, 0), (0, s_pad - S)))

    BT = 8
    while B % BT != 0:
        BT //= 2

    out3 = pl.pallas_call(
        functools.partial(_gate_kernel, s_true=S, needs_mask=(s_pad != S)),
        out_shape=jax.ShapeDtypeStruct((B, C, s_pad), x3.dtype),
        grid=(B // BT,),
        in_specs=[
            pl.BlockSpec((BT, C, s_pad), lambda i: (i, 0, 0)),
            pl.BlockSpec(w1.shape, lambda i: (0, 0)),
            pl.BlockSpec(b1.shape, lambda i: (0, 0)),
            pl.BlockSpec(w2.shape, lambda i: (0, 0)),
            pl.BlockSpec(b2.shape, lambda i: (0, 0)),
        ],
        out_specs=pl.BlockSpec((BT, C, s_pad), lambda i: (i, 0, 0)),
        compiler_params=pltpu.CompilerParams(
            dimension_semantics=("parallel",)
        ),
    )(x3, w1, b1, w2, b2)

    return out3[:, :, :S]


def kernel(x, w1, b1, w2, b2):
    B, C, D, H, W = x.shape
    S = D * H * W
    x3 = x.reshape(B, C, S)
    if S % 8 == 0:
        out3 = _channel_gate_native(x3, w1, b1, w2, b2, S)
    else:
        out3 = _channel_gate_padded(x3, w1, b1, w2, b2, S)
    return out3.reshape(B, C, D, H, W)


# speedup vs baseline: 5.8980x; 1.0053x over previous
import functools

import jax
import jax.numpy as jnp
from jax.experimental import pallas as pl
from jax.experimental.pallas import tpu as pltpu


def _pool_kernel_t(x_ref, sum_ref, max_ref):
    k = pl.program_id(1)
    x = x_ref[...].astype(jnp.float32)
    ps = jnp.sum(x, axis=1)
    pm = jnp.max(x, axis=1)

    @pl.when(k == 0)
    def _():
        sum_ref[0] = ps
        max_ref[0] = pm

    @pl.when(k != 0)
    def _():
        sum_ref[0] = sum_ref[0] + ps
        max_ref[0] = jnp.maximum(max_ref[0], pm)


def _apply_kernel_t(x_ref, psum_ref, pmax_ref, w1t_ref, b1_ref, w2_ref,
                    b2_ref, o_ref, *, inv_s):
    s = jnp.sum(psum_ref[...], axis=0)
    m = jnp.max(pmax_ref[...], axis=0)
    avg = s * inv_s

    w1t = w1t_ref[...]
    w2 = w2_ref[...]
    b1 = b1_ref[...].reshape(-1, 1)
    b2 = b2_ref[...].reshape(-1, 1)

    def mlp(p):
        h = jax.lax.dot_general(
            w1t, p, (((1,), (0,)), ((), ())),
            preferred_element_type=jnp.float32)
        h = jnp.maximum(h + b1, 0.0)
        o = jax.lax.dot_general(
            w2, h, (((0,), (0,)), ((), ())),
            preferred_element_type=jnp.float32)
        return o + b2

    scale = jax.nn.sigmoid(mlp(avg) + mlp(m))
    o_ref[...] = (x_ref[...] * scale[:, None, :].astype(o_ref.dtype))


def _channel_gate_native(x3, w1, b1, w2, b2, S):
    B, C, _ = x3.shape
    xT = jnp.transpose(x3, (1, 2, 0))

    ST = next(t for t in (512, 256, 128, 64, 32, 16, 8) if S % t == 0)
    N = S // ST
    P = 2 if N % 2 == 0 else 1
    K = N // P

    STp = next(t for t in (1024, 512, 256, 128, 64, 32, 16, 8)
               if S % t == 0)
    Np = S // STp
    Pp = 2 if Np % 2 == 0 else 1
    Kp = Np // Pp

    psum, pmax = pl.pallas_call(
        _pool_kernel_t,
        out_shape=(
            jax.ShapeDtypeStruct((Pp, C, B), jnp.float32),
            jax.ShapeDtypeStruct((Pp, C, B), jnp.float32),
        ),
        grid=(Pp, Kp),
        in_specs=[pl.BlockSpec((C, STp, B), lambda p, k: (0, p * Kp + k, 0))],
        out_specs=(
            pl.BlockSpec((1, C, B), lambda p, k: (p, 0, 0)),
            pl.BlockSpec((1, C, B), lambda p, k: (p, 0, 0)),
        ),
        compiler_params=pltpu.CompilerParams(
            dimension_semantics=("parallel", "arbitrary")
        ),
    )(xT)

    w1t = jnp.transpose(w1)

    outT = pl.pallas_call(
        functools.partial(_apply_kernel_t, inv_s=1.0 / S),
        out_shape=jax.ShapeDtypeStruct((C, S, B), x3.dtype),
        grid=(P, K),
        in_specs=[
            pl.BlockSpec((C, ST, B), lambda p, k: (0, p * K + k, 0)),
            pl.BlockSpec((Pp, C, B), lambda p, k: (0, 0, 0)),
            pl.BlockSpec((Pp, C, B), lambda p, k: (0, 0, 0)),
            pl.BlockSpec(w1t.shape, lambda p, k: (0, 0)),
            pl.BlockSpec(b1.shape, lambda p, k: (0, 0)),
            pl.BlockSpec(w2.shape, lambda p, k: (0, 0)),
            pl.BlockSpec(b2.shape, lambda p, k: (0, 0)),
        ],
        out_specs=pl.BlockSpec((C, ST, B), lambda p, k: (0, p * K + k, 0)),
        compiler_params=pltpu.CompilerParams(
            dimension_semantics=("parallel", "parallel")
        ),
    )(xT, psum, pmax, w1t, b1, w2, b2)

    return jnp.transpose(outT, (2, 0, 1))


def _gate_kernel(x_ref, w1_ref, b1_ref, w2_ref, b2_ref, o_ref, *,
                 s_true, needs_mask):
    x = x_ref[...].astype(jnp.float32)

    if needs_mask:
        lane = jax.lax.broadcasted_iota(jnp.int32, x.shape, 2)
        x_for_max = jnp.where(lane < s_true, x, -jnp.inf)
    else:
        x_for_max = x

    avg = jnp.sum(x, axis=-1) * (1.0 / s_true)
    mx = jnp.max(x_for_max, axis=-1)

    def mlp(p):
        h = jnp.maximum(
            jnp.dot(p, w1_ref[...], preferred_element_type=jnp.float32)
            + b1_ref[...], 0.0)
        return jnp.dot(h, w2_ref[...],
                       preferred_element_type=jnp.float32) + b2_ref[...]

    scale = jax.nn.sigmoid(mlp(avg) + mlp(mx))
    o_ref[...] = (x * scale[:, :, None]).astype(o_ref.dtype)


def _channel_gate_padded(x3, w1, b1, w2, b2, S):
    B, C, _ = x3.shape
    LANE = 128
    s_pad = -(-S // LANE) * LANE
    if s_pad != S:
        x3 = jnp.pad(x3, ((0, 0), (0, 0), (0, s_pad - S)))

    BT = 8
    while B % BT != 0:
        BT //= 2

    out3 = pl.pallas_call(
        functools.partial(_gate_kernel, s_true=S, needs_mask=(s_pad != S)),
        out_shape=jax.ShapeDtypeStruct((B, C, s_pad), x3.dtype),
        grid=(B // BT,),
        in_specs=[
            pl.BlockSpec((BT, C, s_pad), lambda i: (i, 0, 0)),
            pl.BlockSpec(w1.shape, lambda i: (0, 0)),
            pl.BlockSpec(b1.shape, lambda i: (0, 0)),
            pl.BlockSpec(w2.shape, lambda i: (0, 0)),
            pl.BlockSpec(b2.shape, lambda i: (0, 0)),
        ],
        out_specs=pl.BlockSpec((BT, C, s_pad), lambda i: (i, 0, 0)),
        compiler_params=pltpu.CompilerParams(
            dimension_semantics=("parallel",)
        ),
    )(x3, w1, b1, w2, b2)

    return out3[:, :, :S]


def kernel(x, w1, b1, w2, b2):
    B, C, D, H, W = x.shape
    S = D * H * W
    x3 = x.reshape(B, C, S)
    if S % 8 == 0:
        out3 = _channel_gate_native(x3, w1, b1, w2, b2, S)
    else:
        out3 = _channel_gate_padded(x3, w1, b1, w2, b2, S)
    return out3.reshape(B, C, D, H, W)


# speedup vs baseline: 7.9949x; 1.3555x over previous
import functools

import jax
import jax.numpy as jnp
from jax.experimental import pallas as pl
from jax.experimental.pallas import tpu as pltpu


def _fused_kernel_t(x_ref, w1t_ref, b1_ref, w2_ref, b2_ref, o_ref,
                    stash_ref, accs_ref, accm_ref, scale_ref, *,
                    inv_s, st, k_last):
    ph = pl.program_id(0)
    k = pl.program_id(1)

    @pl.when(ph == 0)
    def _():
        x = x_ref[...].astype(jnp.float32)
        ps = jnp.sum(x, axis=1)
        pm = jnp.max(x, axis=1)

        @pl.when(k == 0)
        def _():
            accs_ref[...] = ps
            accm_ref[...] = pm

        @pl.when(k != 0)
        def _():
            accs_ref[...] = accs_ref[...] + ps
            accm_ref[...] = jnp.maximum(accm_ref[...], pm)

        stash_ref[:, pl.ds(k * st, st), :] = x.astype(jnp.bfloat16)

        @pl.when(k == k_last)
        def _():
            avg = accs_ref[...] * inv_s
            mx = accm_ref[...]
            w1t = w1t_ref[...]
            w2 = w2_ref[...]
            b1 = b1_ref[...].reshape(-1, 1)
            b2 = b2_ref[...].reshape(-1, 1)

            def mlp(p):
                h = jax.lax.dot_general(
                    w1t, p, (((1,), (0,)), ((), ())),
                    preferred_element_type=jnp.float32)
                h = jnp.maximum(h + b1, 0.0)
                o = jax.lax.dot_general(
                    w2, h, (((0,), (0,)), ((), ())),
                    preferred_element_type=jnp.float32)
                return o + b2

            scale_ref[...] = jax.nn.sigmoid(mlp(avg) + mlp(mx))

    @pl.when(ph == 1)
    def _():
        xb = stash_ref[:, pl.ds(k * st, st), :].astype(jnp.float32)
        o_ref[...] = (xb * scale_ref[...][:, None, :]).astype(o_ref.dtype)


def _channel_gate_fused(x3, w1, b1, w2, b2, S):
    B, C, _ = x3.shape
    xT = jnp.transpose(x3, (1, 2, 0))
    w1t = jnp.transpose(w1)

    ST = next(t for t in (256, 128, 64, 32, 16, 8) if S % t == 0)
    K = S // ST

    outT = pl.pallas_call(
        functools.partial(_fused_kernel_t, inv_s=1.0 / S, st=ST, k_last=K - 1),
        out_shape=jax.ShapeDtypeStruct((C, S, B), x3.dtype),
        grid=(2, K),
        in_specs=[
            pl.BlockSpec((C, ST, B),
                         lambda ph, k: (0, jnp.where(ph == 0, k, K - 1), 0)),
            pl.BlockSpec(w1t.shape, lambda ph, k: (0, 0)),
            pl.BlockSpec(b1.shape, lambda ph, k: (0, 0)),
            pl.BlockSpec(w2.shape, lambda ph, k: (0, 0)),
            pl.BlockSpec(b2.shape, lambda ph, k: (0, 0)),
        ],
        out_specs=pl.BlockSpec((C, ST, B),
                               lambda ph, k: (0, jnp.where(ph == 0, 0, k), 0)),
        scratch_shapes=[
            pltpu.VMEM((C, S, B), jnp.bfloat16),
            pltpu.VMEM((C, B), jnp.float32),
            pltpu.VMEM((C, B), jnp.float32),
            pltpu.VMEM((C, B), jnp.float32),
        ],
        compiler_params=pltpu.CompilerParams(
            dimension_semantics=("arbitrary", "arbitrary")
        ),
    )(xT, w1t, b1, w2, b2)

    return jnp.transpose(outT, (2, 0, 1))


def _pool_kernel_t(x_ref, sum_ref, max_ref):
    k = pl.program_id(1)
    x = x_ref[...].astype(jnp.float32)
    ps = jnp.sum(x, axis=1)
    pm = jnp.max(x, axis=1)

    @pl.when(k == 0)
    def _():
        sum_ref[0] = ps
        max_ref[0] = pm

    @pl.when(k != 0)
    def _():
        sum_ref[0] = sum_ref[0] + ps
        max_ref[0] = jnp.maximum(max_ref[0], pm)


def _apply_kernel_t(x_ref, psum_ref, pmax_ref, w1t_ref, b1_ref, w2_ref,
                    b2_ref, o_ref, *, inv_s):
    s = jnp.sum(psum_ref[...], axis=0)
    m = jnp.max(pmax_ref[...], axis=0)
    avg = s * inv_s

    w1t = w1t_ref[...]
    w2 = w2_ref[...]
    b1 = b1_ref[...].reshape(-1, 1)
    b2 = b2_ref[...].reshape(-1, 1)

    def mlp(p):
        h = jax.lax.dot_general(
            w1t, p, (((1,), (0,)), ((), ())),
            preferred_element_type=jnp.float32)
        h = jnp.maximum(h + b1, 0.0)
        o = jax.lax.dot_general(
            w2, h, (((0,), (0,)), ((), ())),
            preferred_element_type=jnp.float32)
        return o + b2

    scale = jax.nn.sigmoid(mlp(avg) + mlp(m))
    o_ref[...] = (x_ref[...] * scale[:, None, :].astype(o_ref.dtype))


def _channel_gate_native(x3, w1, b1, w2, b2, S):
    B, C, _ = x3.shape
    xT = jnp.transpose(x3, (1, 2, 0))

    ST = next(t for t in (512, 256, 128, 64, 32, 16, 8) if S % t == 0)
    N = S // ST
    P = 2 if N % 2 == 0 else 1
    K = N // P

    STp = next(t for t in (1024, 512, 256, 128, 64, 32, 16, 8)
               if S % t == 0)
    Np = S // STp
    Pp = 2 if Np % 2 == 0 else 1
    Kp = Np // Pp

    psum, pmax = pl.pallas_call(
        _pool_kernel_t,
        out_shape=(
            jax.ShapeDtypeStruct((Pp, C, B), jnp.float32),
            jax.ShapeDtypeStruct((Pp, C, B), jnp.float32),
        ),
        grid=(Pp, Kp),
        in_specs=[pl.BlockSpec((C, STp, B), lambda p, k: (0, p * Kp + k, 0))],
        out_specs=(
            pl.BlockSpec((1, C, B), lambda p, k: (p, 0, 0)),
            pl.BlockSpec((1, C, B), lambda p, k: (p, 0, 0)),
        ),
        compiler_params=pltpu.CompilerParams(
            dimension_semantics=("parallel", "arbitrary")
        ),
    )(xT)

    w1t = jnp.transpose(w1)

    outT = pl.pallas_call(
        functools.partial(_apply_kernel_t, inv_s=1.0 / S),
        out_shape=jax.ShapeDtypeStruct((C, S, B), x3.dtype),
        grid=(P, K),
        in_specs=[
            pl.BlockSpec((C, ST, B), lambda p, k: (0, p * K + k, 0)),
            pl.BlockSpec((Pp, C, B), lambda p, k: (0, 0, 0)),
            pl.BlockSpec((Pp, C, B), lambda p, k: (0, 0, 0)),
            pl.BlockSpec(w1t.shape, lambda p, k: (0, 0)),
            pl.BlockSpec(b1.shape, lambda p, k: (0, 0)),
            pl.BlockSpec(w2.shape, lambda p, k: (0, 0)),
            pl.BlockSpec(b2.shape, lambda p, k: (0, 0)),
        ],
        out_specs=pl.BlockSpec((C, ST, B), lambda p, k: (0, p * K + k, 0)),
        compiler_params=pltpu.CompilerParams(
            dimension_semantics=("parallel", "parallel")
        ),
    )(xT, psum, pmax, w1t, b1, w2, b2)

    return jnp.transpose(outT, (2, 0, 1))


def _gate_kernel(x_ref, w1_ref, b1_ref, w2_ref, b2_ref, o_ref, *,
                 s_true, needs_mask):
    x = x_ref[...].astype(jnp.float32)

    if needs_mask:
        lane = jax.lax.broadcasted_iota(jnp.int32, x.shape, 2)
        x_for_max = jnp.where(lane < s_true, x, -jnp.inf)
    else:
        x_for_max = x

    avg = jnp.sum(x, axis=-1) * (1.0 / s_true)
    mx = jnp.max(x_for_max, axis=-1)

    def mlp(p):
        h = jnp.maximum(
            jnp.dot(p, w1_ref[...], preferred_element_type=jnp.float32)
            + b1_ref[...], 0.0)
        return jnp.dot(h, w2_ref[...],
                       preferred_element_type=jnp.float32) + b2_ref[...]

    scale = jax.nn.sigmoid(mlp(avg) + mlp(mx))
    o_ref[...] = (x * scale[:, :, None]).astype(o_ref.dtype)


def _channel_gate_padded(x3, w1, b1, w2, b2, S):
    B, C, _ = x3.shape
    LANE = 128
    s_pad = -(-S // LANE) * LANE
    if s_pad != S:
        x3 = jnp.pad(x3, ((0, 0), (0, 0), (0, s_pad - S)))

    BT = 8
    while B % BT != 0:
        BT //= 2

    out3 = pl.pallas_call(
        functools.partial(_gate_kernel, s_true=S, needs_mask=(s_pad != S)),
        out_shape=jax.ShapeDtypeStruct((B, C, s_pad), x3.dtype),
        grid=(B // BT,),
        in_specs=[
            pl.BlockSpec((BT, C, s_pad), lambda i: (i, 0, 0)),
            pl.BlockSpec(w1.shape, lambda i: (0, 0)),
            pl.BlockSpec(b1.shape, lambda i: (0, 0)),
            pl.BlockSpec(w2.shape, lambda i: (0, 0)),
            pl.BlockSpec(b2.shape, lambda i: (0, 0)),
        ],
        out_specs=pl.BlockSpec((BT, C, s_pad), lambda i: (i, 0, 0)),
        compiler_params=pltpu.CompilerParams(
            dimension_semantics=("parallel",)
        ),
    )(x3, w1, b1, w2, b2)

    return out3[:, :, :S]


def kernel(x, w1, b1, w2, b2):
    B, C, D, H, W = x.shape
    S = D * H * W
    x3 = x.reshape(B, C, S)
    if S % 8 == 0:
        lanes = -(-B // 128) * 128
        st = next(t for t in (256, 128, 64, 32, 16, 8) if S % t == 0)
        vmem_bytes = C * S * lanes * 2 + 4 * C * st * lanes * 4
        if vmem_bytes <= 52 * 1024 * 1024:
            out3 = _channel_gate_fused(x3, w1, b1, w2, b2, S)
        else:
            out3 = _channel_gate_native(x3, w1, b1, w2, b2, S)
    else:
        out3 = _channel_gate_padded(x3, w1, b1, w2, b2, S)
    return out3.reshape(B, C, D, H, W)
